# Initial kernel scaffold; baseline (speedup 1.0000x reference)
import jax
import jax.numpy as jnp
from jax.experimental import pallas as pl


def kernel(g_x, g_pe, g_edge_index, g_spd, g_root_n_index, token_list, token_pe, shared_edge_weight, edge_weight_param, de, drop_e):
    raise NotImplementedError("write your pallas kernel here")


# speedup vs baseline: 8.2991x; 8.2991x over previous
import functools

import jax
import jax.numpy as jnp
from jax import lax
from jax.experimental import pallas as pl
from jax.experimental.pallas import tpu as pltpu
from jax.experimental.pallas import tpu_sc as plsc

_T = 10
_N = 10000
_E = 320000
_LEN_P1 = 5
_INNER = _T * (_T - 1)
_CROSS = _T * _N
_EDGES = _INNER + _E + 2 * _CROSS

_S1 = _INNER
_S2 = _S1 + _E
_S3 = _S2 + _CROSS

_NWORKERS = 32
_CHUNK = 16256
_TAIL = _EDGES - (_NWORKERS - 1) * _CHUNK
_WV_ITERS = _N // 16
_EW_ITERS = _CHUNK // 16


def _xpe_body(tok_ref, gx_ref, tpe_ref, gpe_ref, x_ref, pe_ref):
    x_ref[0:_T, :] = tok_ref[...]
    x_ref[_T:, :] = gx_ref[...]
    pe_ref[0:_T, :] = tpe_ref[...]
    pe_ref[_T:, :] = gpe_ref[...]


def _ei_body(gei_ref, ei_ref):
    k = lax.broadcasted_iota(jnp.int32, (1, _INNER), 1)
    i = k // (_T - 1)
    r = k % (_T - 1)
    j = r + (r >= i).astype(jnp.int32)
    ei_ref[0:1, 0:_S1] = i
    ei_ref[1:2, 0:_S1] = j
    ei_ref[:, _S1:_S2] = gei_ref[...] + _T
    c = lax.broadcasted_iota(jnp.int32, (1, _CROSS), 1)
    ei_ref[0:1, _S2:_S3] = c // _N
    ei_ref[1:2, _S2:_S3] = c % _N + _T
    ei_ref[0:1, _S3:_EDGES] = c // _T + _T
    ei_ref[1:2, _S3:_EDGES] = c % _T


def _ew_body(spd_hbm, small_hbm, out_hbm, spd_v, small_v, wv_v, out_v):
    wid = lax.axis_index("s") * 2 + lax.axis_index("c")
    pltpu.sync_copy(spd_hbm, spd_v)
    pltpu.sync_copy(small_hbm, small_v)

    def wv_step(it, carry):
        s = spd_v[pl.ds(it * 16, 16)]
        s = jnp.minimum(jnp.maximum(s, 0), _LEN_P1 - 1)
        wv_v[pl.ds(it * 16, 16)] = plsc.load_gather(small_v, [s])
        return carry

    lax.fori_loop(0, _WV_ITERS, wv_step, 0)

    shared_vec = plsc.load_gather(
        small_v, [jnp.full((16,), _LEN_P1, jnp.int32)])
    ones = jnp.ones((16,), jnp.float32)
    lane = lax.iota(jnp.int32, 16)
    lo = wid * _CHUNK

    def ew_step(it, carry):
        j = lo + it * 16 + lane
        i3 = lax.rem(j - _S2, _N)
        i3 = jnp.minimum(jnp.maximum(i3, 0), _N - 1)
        i4 = lax.div(j - _S3, _T)
        i4 = jnp.minimum(jnp.maximum(i4, 0), _N - 1)
        g3 = plsc.load_gather(wv_v, [i3])
        g4 = plsc.load_gather(wv_v, [i4])
        v = jnp.where(j < _S2,
                      jnp.where(j < _S1, shared_vec, ones),
                      jnp.where(j < _S3, g3, g4))
        out_v[pl.ds(it * 16, 16)] = v
        return carry

    lax.fori_loop(0, _EW_ITERS, ew_step, 0)

    @pl.when(wid < _NWORKERS - 1)
    def _():
        pltpu.sync_copy(out_v, out_hbm.at[pl.ds(lo, _CHUNK)])

    @pl.when(wid == _NWORKERS - 1)
    def _():
        pltpu.sync_copy(out_v.at[pl.ds(0, _TAIL)],
                        out_hbm.at[pl.ds((_NWORKERS - 1) * _CHUNK, _TAIL)])


@functools.cache
def _ew_call():
    return functools.partial(
        pl.kernel,
        mesh=plsc.VectorSubcoreMesh(core_axis_name="c", subcore_axis_name="s"),
        compiler_params=pltpu.CompilerParams(needs_layout_passes=False),
        out_type=jax.ShapeDtypeStruct((_EDGES,), jnp.float32),
        scratch_types=[
            pltpu.VMEM((_N,), jnp.int32),
            pltpu.VMEM((16,), jnp.float32),
            pltpu.VMEM((_N,), jnp.float32),
            pltpu.VMEM((_CHUNK,), jnp.float32),
        ],
    )(_ew_body)


def kernel(g_x, g_pe, g_edge_index, g_spd, g_root_n_index, token_list,
           token_pe, shared_edge_weight, edge_weight_param, de, drop_e):
    x, pe = pl.pallas_call(
        _xpe_body,
        out_shape=(
            jax.ShapeDtypeStruct((_T + _N, 128), jnp.float32),
            jax.ShapeDtypeStruct((_T + _N, 32), jnp.float32),
        ),
    )(token_list, g_x, token_pe, g_pe)

    ei = pl.pallas_call(
        _ei_body,
        out_shape=jax.ShapeDtypeStruct((2, _EDGES), jnp.int32),
    )(g_edge_index)

    small = jnp.concatenate([
        edge_weight_param.astype(jnp.float32),
        jnp.reshape(shared_edge_weight.astype(jnp.float32), (1,)),
        jnp.zeros((10,), jnp.float32),
    ])
    ew = _ew_call()(g_spd, small)

    root = g_root_n_index + _T
    return (x, pe, ei, ew, root)


# speedup vs baseline: 21.6735x; 2.6116x over previous
import functools

import jax
import jax.numpy as jnp
from jax import lax
from jax.experimental import pallas as pl
from jax.experimental.pallas import tpu as pltpu
from jax.experimental.pallas import tpu_sc as plsc

_T = 10
_N = 10000
_E = 320000
_LEN_P1 = 5
_INNER = _T * (_T - 1)
_CROSS = _T * _N
_EDGES = _INNER + _E + 2 * _CROSS

_S1 = _INNER
_S2 = _S1 + _E
_S3 = _S2 + _CROSS

_NWORKERS = 32
_HEAD = 96
_ONES_PER_W = 10000
_TILE_A = _S2 - 2
_REP_A = _S3 - 2
_REP_PER_W = 3128
_REP_BUF = 3136
_REP_TAIL = _EDGES - (_REP_A + 31 * _REP_PER_W)
_WV_PAD = 10016
_PIECE = 640


def _xpe_body(tok_ref, gx_ref, tpe_ref, gpe_ref, x_ref, pe_ref):
    x_ref[0:_T, :] = tok_ref[...]
    x_ref[_T:, :] = gx_ref[...]
    pe_ref[0:_T, :] = tpe_ref[...]
    pe_ref[_T:, :] = gpe_ref[...]


def _ei_body(gei_ref, ei_ref):
    k = lax.broadcasted_iota(jnp.int32, (1, _INNER), 1)
    i = k // (_T - 1)
    r = k % (_T - 1)
    j = r + (r >= i).astype(jnp.int32)
    ei_ref[0:1, 0:_S1] = i
    ei_ref[1:2, 0:_S1] = j
    ei_ref[:, _S1:_S2] = gei_ref[...] + _T
    c = lax.broadcasted_iota(jnp.int32, (1, _CROSS), 1)
    ei_ref[0:1, _S2:_S3] = c // _N
    ei_ref[1:2, _S2:_S3] = c % _N + _T
    ei_ref[0:1, _S3:_EDGES] = c // _T + _T
    ei_ref[1:2, _S3:_EDGES] = c % _T


def _ew_body(spd_hbm, small_hbm, ones_hbm, out_hbm,
             spd_v, small_v, wv_pad_v, piece_v, rep_v, ones_v, wv_sh):
    s = lax.axis_index("s")
    c = lax.axis_index("c")
    wid = s * 2 + c
    lane = lax.iota(jnp.int32, 16)
    ones = jnp.ones((16,), jnp.float32)

    pltpu.sync_copy(spd_hbm, spd_v)
    pltpu.sync_copy(small_hbm, small_v)

    base_k = s * _PIECE

    def piece_step(it, carry):
        k = base_k + it * 16 + lane
        idx = k - 2
        idx = jnp.where(idx < 0, idx + _N,
                        jnp.where(idx >= _N, idx - _N, idx))
        sp = plsc.load_gather(spd_v, [idx])
        sp = jnp.minimum(jnp.maximum(sp, 0), _LEN_P1 - 1)
        piece_v[pl.ds(it * 16, 16)] = plsc.load_gather(small_v, [sp])
        return carry

    @pl.when(s < 15)
    def _():
        lax.fori_loop(0, _PIECE // 16, piece_step, 0, unroll=8)
        pltpu.sync_copy(piece_v, wv_sh.at[pl.ds(base_k, _PIECE)])

    @pl.when(s == 15)
    def _():
        lax.fori_loop(0, (_WV_PAD - 15 * _PIECE) // 16, piece_step, 0,
                      unroll=8)
        pltpu.sync_copy(piece_v.at[pl.ds(0, _WV_PAD - 15 * _PIECE)],
                        wv_sh.at[pl.ds(base_k, _WV_PAD - 15 * _PIECE)])

    pltpu.sync_copy(ones_hbm, ones_v)
    plsc.subcore_barrier()
    pltpu.sync_copy(wv_sh, wv_pad_v)

    @pl.when(wid == 0)
    def _():
        shared_vec = plsc.load_gather(
            small_v, [jnp.full((16,), _LEN_P1, jnp.int32)])

        def head_step(it, carry):
            j = it * 16 + lane
            piece_v[pl.ds(it * 16, 16)] = jnp.where(j < _S1, shared_vec, ones)
            return carry

        lax.fori_loop(0, _HEAD // 16, head_step, 0, unroll=6)
        pltpu.sync_copy(piece_v.at[pl.ds(0, _HEAD)],
                        out_hbm.at[pl.ds(0, _HEAD)])

    ones_lo = _HEAD + wid * _ONES_PER_W

    @pl.when(wid < _NWORKERS - 1)
    def _():
        pltpu.sync_copy(ones_v.at[pl.ds(0, _ONES_PER_W)],
                        out_hbm.at[pl.ds(ones_lo, _ONES_PER_W)])

    @pl.when(wid == _NWORKERS - 1)
    def _():
        pltpu.sync_copy(ones_v.at[pl.ds(0, _TILE_A - _HEAD - 31 * _ONES_PER_W)],
                        out_hbm.at[pl.ds(ones_lo,
                                         _TILE_A - _HEAD - 31 * _ONES_PER_W)])

    @pl.when(wid == 0)
    def _():
        pltpu.sync_copy(wv_pad_v.at[pl.ds(8, _N - 6)],
                        out_hbm.at[pl.ds(_TILE_A + 8, _N - 6)])

    @pl.when(jnp.logical_and(wid >= 1, wid <= 9))
    def _():
        pltpu.sync_copy(wv_pad_v.at[pl.ds(0, _N + 2)],
                        out_hbm.at[pl.ds(_TILE_A + wid * _N, _N + 2)])

    @pl.when(wid == 10)
    def _():
        v = wv_pad_v[pl.ds(0, 16)]
        piece_v[pl.ds(0, 16)] = jnp.where(lane < 2, ones, v)
        pltpu.sync_copy(piece_v.at[pl.ds(0, 8)],
                        out_hbm.at[pl.ds(_TILE_A, 8)])

    rep_lo = _REP_A + wid * _REP_PER_W

    def rep_step(it, carry):
        j = rep_lo + it * 16 + lane
        i4 = jnp.where(j < _S3, j - (_REP_A - _N + 2), lax.div(j - _S3, _T))
        i4 = jnp.minimum(jnp.maximum(i4, 0), _N - 1)
        rep_v[pl.ds(it * 16, 16)] = plsc.load_gather(wv_pad_v, [i4 + 2])
        return carry

    lax.fori_loop(0, _REP_BUF // 16, rep_step, 0, unroll=8)

    @pl.when(wid < _NWORKERS - 1)
    def _():
        pltpu.sync_copy(rep_v.at[pl.ds(0, _REP_PER_W)],
                        out_hbm.at[pl.ds(rep_lo, _REP_PER_W)])

    @pl.when(wid == _NWORKERS - 1)
    def _():
        pltpu.sync_copy(rep_v.at[pl.ds(0, _REP_TAIL)],
                        out_hbm.at[pl.ds(rep_lo, _REP_TAIL)])


@functools.cache
def _ew_call():
    return functools.partial(
        pl.kernel,
        mesh=plsc.VectorSubcoreMesh(core_axis_name="c", subcore_axis_name="s"),
        compiler_params=pltpu.CompilerParams(needs_layout_passes=False),
        out_type=jax.ShapeDtypeStruct((_EDGES,), jnp.float32),
        scratch_types=[
            pltpu.VMEM((_N,), jnp.int32),
            pltpu.VMEM((16,), jnp.float32),
            pltpu.VMEM((_WV_PAD,), jnp.float32),
            pltpu.VMEM((_PIECE,), jnp.float32),
            pltpu.VMEM((_REP_BUF,), jnp.float32),
            pltpu.VMEM((_ONES_PER_W + 16,), jnp.float32),
            pltpu.VMEM_SHARED((_WV_PAD,), jnp.float32),
        ],
    )(_ew_body)


def kernel(g_x, g_pe, g_edge_index, g_spd, g_root_n_index, token_list,
           token_pe, shared_edge_weight, edge_weight_param, de, drop_e):
    x, pe = pl.pallas_call(
        _xpe_body,
        out_shape=(
            jax.ShapeDtypeStruct((_T + _N, 128), jnp.float32),
            jax.ShapeDtypeStruct((_T + _N, 32), jnp.float32),
        ),
    )(token_list, g_x, token_pe, g_pe)

    ei = pl.pallas_call(
        _ei_body,
        out_shape=jax.ShapeDtypeStruct((2, _EDGES), jnp.int32),
    )(g_edge_index)

    small = jnp.concatenate([
        edge_weight_param.astype(jnp.float32),
        jnp.reshape(shared_edge_weight.astype(jnp.float32), (1,)),
        jnp.zeros((10,), jnp.float32),
    ])
    ew = _ew_call()(g_spd, small, jnp.ones((_ONES_PER_W + 16,), jnp.float32))

    root = g_root_n_index + _T
    return (x, pe, ei, ew, root)


# speedup vs baseline: 22.1700x; 1.0229x over previous
import functools

import jax
import jax.numpy as jnp
from jax import lax
from jax.experimental import pallas as pl
from jax.experimental.pallas import tpu as pltpu
from jax.experimental.pallas import tpu_sc as plsc

_T = 10
_N = 10000
_E = 320000
_LEN_P1 = 5
_INNER = _T * (_T - 1)
_CROSS = _T * _N
_EDGES = _INNER + _E + 2 * _CROSS

_S1 = _INNER
_S2 = _S1 + _E
_S3 = _S2 + _CROSS

_NWORKERS = 32
_HEAD = 96
_ONES_PER_W = 10000
_TILE_A = _S2 - 2
_REP_A = _S3 - 2
_REP_PER_W = 3128
_REP_BUF = 3136
_REP_TAIL = _EDGES - (_REP_A + 31 * _REP_PER_W)
_WV_PAD = 10016
_PIECE = 640


def _tc_body(tok_ref, gx_ref, tpe_ref, gpe_ref, gei_ref, x_ref, pe_ref,
             ei_ref):
    x_ref[0:_T, :] = tok_ref[...]
    x_ref[_T:, :] = gx_ref[...]
    pe_ref[0:_T, :] = tpe_ref[...]
    pe_ref[_T:, :] = gpe_ref[...]
    k = lax.broadcasted_iota(jnp.int32, (1, _INNER), 1)
    i = k // (_T - 1)
    r = k % (_T - 1)
    j = r + (r >= i).astype(jnp.int32)
    ei_ref[0:1, 0:_S1] = i
    ei_ref[1:2, 0:_S1] = j
    ei_ref[:, _S1:_S2] = gei_ref[...] + _T
    c = lax.broadcasted_iota(jnp.int32, (1, _CROSS), 1)
    ei_ref[0:1, _S2:_S3] = c // _N
    ei_ref[1:2, _S2:_S3] = c % _N + _T
    ei_ref[0:1, _S3:_EDGES] = c // _T + _T
    ei_ref[1:2, _S3:_EDGES] = c % _T


def _ew_body(spd_hbm, combo_hbm, out_hbm,
             spd_v, wv_pad_v, piece_v, rep_v, combo_v, wv_sh):
    s = lax.axis_index("s")
    c = lax.axis_index("c")
    wid = s * 2 + c
    lane = lax.iota(jnp.int32, 16)
    ones = jnp.ones((16,), jnp.float32)

    pltpu.sync_copy(spd_hbm, spd_v)
    pltpu.sync_copy(combo_hbm, combo_v)

    base_k = s * _PIECE

    def piece_step(it, carry):
        k = base_k + it * 16 + lane
        idx = k - 2
        idx = jnp.where(idx < 0, idx + _N,
                        jnp.where(idx >= _N, idx - _N, idx))
        sp = plsc.load_gather(spd_v, [idx])
        sp = jnp.minimum(jnp.maximum(sp, 0), _LEN_P1 - 1)
        piece_v[pl.ds(it * 16, 16)] = plsc.load_gather(combo_v, [_N + sp])
        return carry

    @pl.when(s < 15)
    def _():
        lax.fori_loop(0, _PIECE // 16, piece_step, 0, unroll=8)
        pltpu.sync_copy(piece_v, wv_sh.at[pl.ds(base_k, _PIECE)])

    @pl.when(s == 15)
    def _():
        lax.fori_loop(0, (_WV_PAD - 15 * _PIECE) // 16, piece_step, 0,
                      unroll=8)
        pltpu.sync_copy(piece_v.at[pl.ds(0, _WV_PAD - 15 * _PIECE)],
                        wv_sh.at[pl.ds(base_k, _WV_PAD - 15 * _PIECE)])

    @pl.when(wid == 0)
    def _():
        shared_vec = plsc.load_gather(
            combo_v, [jnp.full((16,), _N + _LEN_P1, jnp.int32)])

        def head_step(it, carry):
            j = it * 16 + lane
            piece_v[pl.ds(it * 16, 16)] = jnp.where(j < _S1, shared_vec, ones)
            return carry

        lax.fori_loop(0, _HEAD // 16, head_step, 0, unroll=6)
        pltpu.sync_copy(piece_v.at[pl.ds(0, _HEAD)],
                        out_hbm.at[pl.ds(0, _HEAD)])

    ones_lo = jnp.where(wid < _NWORKERS - 1, _HEAD + wid * _ONES_PER_W,
                        _TILE_A - _ONES_PER_W)
    pltpu.sync_copy(combo_v.at[pl.ds(0, _ONES_PER_W)],
                    out_hbm.at[pl.ds(ones_lo, _ONES_PER_W)])

    plsc.subcore_barrier()
    pltpu.sync_copy(wv_sh, wv_pad_v)

    @pl.when(wid == 0)
    def _():
        pltpu.sync_copy(wv_pad_v.at[pl.ds(8, _N - 6)],
                        out_hbm.at[pl.ds(_TILE_A + 8, _N - 6)])

    @pl.when(jnp.logical_and(wid >= 1, wid <= 9))
    def _():
        pltpu.sync_copy(wv_pad_v.at[pl.ds(0, _N + 2)],
                        out_hbm.at[pl.ds(_TILE_A + wid * _N, _N + 2)])

    @pl.when(wid == 10)
    def _():
        v = wv_pad_v[pl.ds(0, 16)]
        piece_v[pl.ds(0, 16)] = jnp.where(lane < 2, ones, v)
        pltpu.sync_copy(piece_v.at[pl.ds(0, 8)],
                        out_hbm.at[pl.ds(_TILE_A, 8)])

    rep_lo = _REP_A + wid * _REP_PER_W

    def rep_step(it, carry):
        j = rep_lo + it * 16 + lane
        i4 = jnp.where(j < _S3, j - (_REP_A - _N + 2), lax.div(j - _S3, _T))
        i4 = jnp.minimum(jnp.maximum(i4, 0), _N - 1)
        rep_v[pl.ds(it * 16, 16)] = plsc.load_gather(wv_pad_v, [i4 + 2])
        return carry

    lax.fori_loop(0, _REP_BUF // 16, rep_step, 0, unroll=8)

    @pl.when(wid < _NWORKERS - 1)
    def _():
        pltpu.sync_copy(rep_v.at[pl.ds(0, _REP_PER_W)],
                        out_hbm.at[pl.ds(rep_lo, _REP_PER_W)])

    @pl.when(wid == _NWORKERS - 1)
    def _():
        pltpu.sync_copy(rep_v.at[pl.ds(0, _REP_TAIL)],
                        out_hbm.at[pl.ds(rep_lo, _REP_TAIL)])


@functools.cache
def _ew_call():
    return functools.partial(
        pl.kernel,
        mesh=plsc.VectorSubcoreMesh(core_axis_name="c", subcore_axis_name="s"),
        compiler_params=pltpu.CompilerParams(needs_layout_passes=False),
        out_type=jax.ShapeDtypeStruct((_EDGES,), jnp.float32),
        scratch_types=[
            pltpu.VMEM((_N,), jnp.int32),
            pltpu.VMEM((_WV_PAD,), jnp.float32),
            pltpu.VMEM((_PIECE,), jnp.float32),
            pltpu.VMEM((_REP_BUF,), jnp.float32),
            pltpu.VMEM((_WV_PAD,), jnp.float32),
            pltpu.VMEM_SHARED((_WV_PAD,), jnp.float32),
        ],
    )(_ew_body)


def kernel(g_x, g_pe, g_edge_index, g_spd, g_root_n_index, token_list,
           token_pe, shared_edge_weight, edge_weight_param, de, drop_e):
    x, pe, ei = pl.pallas_call(
        _tc_body,
        out_shape=(
            jax.ShapeDtypeStruct((_T + _N, 128), jnp.float32),
            jax.ShapeDtypeStruct((_T + _N, 32), jnp.float32),
            jax.ShapeDtypeStruct((2, _EDGES), jnp.int32),
        ),
    )(token_list, g_x, token_pe, g_pe, g_edge_index)

    combo = jnp.concatenate([
        jnp.ones((_N,), jnp.float32),
        edge_weight_param.astype(jnp.float32),
        jnp.reshape(shared_edge_weight.astype(jnp.float32), (1,)),
        jnp.zeros((10,), jnp.float32),
    ])
    ew = _ew_call()(g_spd, combo)

    root = g_root_n_index + _T
    return (x, pe, ei, ew, root)


# speedup vs baseline: 22.4980x; 1.0148x over previous
import functools

import numpy as np

import jax
import jax.numpy as jnp
from jax import lax
from jax.experimental import pallas as pl
from jax.experimental.pallas import tpu as pltpu
from jax.experimental.pallas import tpu_sc as plsc

_T = 10
_N = 10000
_E = 320000
_LEN_P1 = 5
_INNER = _T * (_T - 1)
_CROSS = _T * _N
_EDGES = _INNER + _E + 2 * _CROSS

_S1 = _INNER
_S2 = _S1 + _E
_S3 = _S2 + _CROSS

_NWORKERS = 32
_WV_PAD = 10016
_PIECE = 640
_W_PER_W = 312
_REP_PER_W = 3136


def _tc_body(tok_ref, gx_ref, tpe_ref, gpe_ref, gei_ref, x_ref, pe_ref,
             ei_ref):
    x_ref[0:_T, :] = tok_ref[...]
    x_ref[_T:, :] = gx_ref[...]
    pe_ref[0:_T, :] = tpe_ref[...]
    pe_ref[_T:, :] = gpe_ref[...]
    k = lax.broadcasted_iota(jnp.int32, (1, _INNER), 1)
    i = k // (_T - 1)
    r = k % (_T - 1)
    j = r + (r >= i).astype(jnp.int32)
    ei_ref[0:1, 0:_S1] = i
    ei_ref[1:2, 0:_S1] = j
    ei_ref[:, _S1:_S2] = gei_ref[...] + _T
    c = lax.broadcasted_iota(jnp.int32, (1, _CROSS), 1)
    ei_ref[0:1, _S2:_S3] = c // _N
    ei_ref[1:2, _S2:_S3] = c % _N + _T
    ei_ref[0:1, _S3:_EDGES] = c // _T + _T
    ei_ref[1:2, _S3:_EDGES] = c % _T


def _ew_tc_body(w_ref, rep_ref, sh_ref, ew_ref):
    idx = lax.broadcasted_iota(jnp.int32, (128,), 0)
    sh = jnp.broadcast_to(sh_ref[...], (128,))
    ew_ref[pl.ds(0, 128)] = jnp.where(idx < _S1, sh, 1.0)
    ew_ref[pl.ds(128, _S2 - 128)] = jnp.ones((_S2 - 128,), jnp.float32)
    for t in range(_T):
        ew_ref[pl.ds(_S2 + t * _N, _N)] = w_ref[...]
    ew_ref[pl.ds(_S3, _CROSS)] = rep_ref[...]


def _ew_sc_body(spd_hbm, par_hbm, w_hbm, rep_hbm,
                spd_v, param_v, piece_v, rep_v, wv_v, wv_sh):
    s = lax.axis_index("s")
    c = lax.axis_index("c")
    wid = s * 2 + c
    lane = lax.iota(jnp.int32, 16)

    pltpu.sync_copy(spd_hbm, spd_v)
    pltpu.sync_copy(par_hbm, param_v)

    base_k = s * _PIECE

    def piece_step(it, carry):
        k = base_k + it * 16 + lane
        idx = jnp.minimum(k, _N - 1)
        sp = plsc.load_gather(spd_v, [idx])
        sp = jnp.minimum(jnp.maximum(sp, 0), _LEN_P1 - 1)
        piece_v[pl.ds(it * 16, 16)] = plsc.load_gather(param_v, [sp])
        return carry

    @pl.when(s < 15)
    def _():
        lax.fori_loop(0, _PIECE // 16, piece_step, 0, unroll=8)
        pltpu.sync_copy(piece_v, wv_sh.at[pl.ds(base_k, _PIECE)])

    @pl.when(s == 15)
    def _():
        lax.fori_loop(0, (_WV_PAD - 15 * _PIECE) // 16, piece_step, 0,
                      unroll=8)
        pltpu.sync_copy(piece_v.at[pl.ds(0, _WV_PAD - 15 * _PIECE)],
                        wv_sh.at[pl.ds(base_k, _WV_PAD - 15 * _PIECE)])

    plsc.subcore_barrier()
    pltpu.sync_copy(wv_sh, wv_v)

    w_lo = wid * _W_PER_W

    @pl.when(wid < _NWORKERS - 1)
    def _():
        pltpu.sync_copy(wv_v.at[pl.ds(w_lo, _W_PER_W)],
                        w_hbm.at[pl.ds(w_lo, _W_PER_W)])

    @pl.when(wid == _NWORKERS - 1)
    def _():
        lo = (_NWORKERS - 1) * _W_PER_W
        pltpu.sync_copy(wv_v.at[pl.ds(lo, _N - lo)],
                        w_hbm.at[pl.ds(lo, _N - lo)])

    rep_lo = wid * _REP_PER_W

    def rep_step(it, carry):
        j = rep_lo + it * 16 + lane
        i4 = jnp.minimum(lax.div(j, _T), _N - 1)
        rep_v[pl.ds(it * 16, 16)] = plsc.load_gather(wv_v, [i4])
        return carry

    @pl.when(wid < _NWORKERS - 1)
    def _():
        lax.fori_loop(0, _REP_PER_W // 16, rep_step, 0, unroll=8)
        pltpu.sync_copy(rep_v.at[pl.ds(0, _REP_PER_W)],
                        rep_hbm.at[pl.ds(rep_lo, _REP_PER_W)])

    @pl.when(wid == _NWORKERS - 1)
    def _():
        tail = _CROSS - (_NWORKERS - 1) * _REP_PER_W
        lax.fori_loop(0, tail // 16, rep_step, 0, unroll=8)
        pltpu.sync_copy(rep_v.at[pl.ds(0, tail)],
                        rep_hbm.at[pl.ds((_NWORKERS - 1) * _REP_PER_W, tail)])


@functools.cache
def _ew_sc_call():
    return functools.partial(
        pl.kernel,
        mesh=plsc.VectorSubcoreMesh(core_axis_name="c", subcore_axis_name="s"),
        compiler_params=pltpu.CompilerParams(needs_layout_passes=False),
        out_type=(
            jax.ShapeDtypeStruct((_N,), jnp.float32),
            jax.ShapeDtypeStruct((_CROSS,), jnp.float32),
        ),
        scratch_types=[
            pltpu.VMEM((_N,), jnp.int32),
            pltpu.VMEM((16,), jnp.float32),
            pltpu.VMEM((_PIECE,), jnp.float32),
            pltpu.VMEM((_REP_PER_W,), jnp.float32),
            pltpu.VMEM((_WV_PAD,), jnp.float32),
            pltpu.VMEM_SHARED((_WV_PAD,), jnp.float32),
        ],
    )(_ew_sc_body)


def kernel(g_x, g_pe, g_edge_index, g_spd, g_root_n_index, token_list,
           token_pe, shared_edge_weight, edge_weight_param, de, drop_e):
    x, pe, ei = pl.pallas_call(
        _tc_body,
        out_shape=(
            jax.ShapeDtypeStruct((_T + _N, 128), jnp.float32),
            jax.ShapeDtypeStruct((_T + _N, 32), jnp.float32),
            jax.ShapeDtypeStruct((2, _EDGES), jnp.int32),
        ),
    )(token_list, g_x, token_pe, g_pe, g_edge_index)

    param16 = jnp.concatenate([
        edge_weight_param.astype(jnp.float32),
        jnp.zeros((16 - _LEN_P1,), jnp.float32),
    ])
    w, rep = _ew_sc_call()(g_spd, param16)

    ew = pl.pallas_call(
        _ew_tc_body,
        out_shape=jax.ShapeDtypeStruct((_EDGES,), jnp.float32),
    )(w, rep, jnp.reshape(shared_edge_weight.astype(jnp.float32), (1,)))

    root = g_root_n_index + _T
    return (x, pe, ei, ew, root)
